# baseline (speedup 1.0000x reference)
import jax
import jax.numpy as jnp
from jax.experimental import pallas as pl
from jax.experimental.pallas import tpu as pltpu

N = 50000
E = 800000
EMB = 32
HID = 64
NCLS = 16
VOC = 512
NREL = 3
NGRAPH = 128
BLK = 1000
NBLK = N // BLK
ECHUNK = 128
EROWS = E // ECHUNK


def _embed_kernel(x_ref, se_ref, ce_ref, pe_ref, h_ref):
    lanes = jax.lax.broadcasted_iota(jnp.int32, (1, VOC), 1)
    for k, tab in ((0, se_ref), (1, ce_ref), (2, pe_ref)):
        idx = x_ref[:, k:k + 1]
        onehot = (idx == lanes).astype(jnp.float32)
        h_ref[:, k * EMB:(k + 1) * EMB] = jnp.dot(
            onehot, tab[...], preferred_element_type=jnp.float32)


def _wgt_kernel(dst_ref, rel_ref, cnt_ref, w_ref):
    cnt_ref[...] = jnp.zeros((N, NREL), jnp.float32)
    lane = jax.lax.broadcasted_iota(jnp.int32, (1, ECHUNK), 1)
    r3 = jax.lax.broadcasted_iota(jnp.int32, (1, NREL), 1)

    def outer(i, _):
        drow = dst_ref[pl.ds(i, 1), :]
        rrow = rel_ref[pl.ds(i, 1), :]

        def inner(j, _):
            m = lane == j
            d = jnp.sum(jnp.where(m, drow, 0))
            r = jnp.sum(jnp.where(m, rrow, 0))
            cnt_ref[pl.ds(d, 1), :] = cnt_ref[pl.ds(d, 1), :] + (
                r3 == r).astype(jnp.float32)
            return 0

        return jax.lax.fori_loop(0, ECHUNK, inner, 0)

    jax.lax.fori_loop(0, EROWS, outer, 0)

    def outer2(i, _):
        drow = dst_ref[pl.ds(i, 1), :]
        rrow = rel_ref[pl.ds(i, 1), :]

        def inner(j, _):
            m = lane == j
            d = jnp.sum(jnp.where(m, drow, 0))
            r = jnp.sum(jnp.where(m, rrow, 0))
            crow = cnt_ref[pl.ds(d, 1), :]
            c = jnp.sum(jnp.where(r3 == r, crow, 0.0))
            w = 1.0 / jnp.maximum(c, 1.0)
            w_ref[pl.ds(i, 1), :] = jnp.where(m, w, w_ref[pl.ds(i, 1), :])
            return 0

        return jax.lax.fori_loop(0, ECHUNK, inner, 0)

    jax.lax.fori_loop(0, EROWS, outer2, 0)


def _mm_kernel(h_ref, w0_ref, w1_ref, w2_ref, wr_ref, b_ref,
               xr01_ref, xr2_ref, init_ref):
    h = h_ref[...]
    xr01_ref[:, 0:HID] = jnp.dot(h, w0_ref[...],
                                 preferred_element_type=jnp.float32)
    xr01_ref[:, HID:2 * HID] = jnp.dot(h, w1_ref[...],
                                       preferred_element_type=jnp.float32)
    xr2_ref[...] = jnp.dot(h, w2_ref[...], preferred_element_type=jnp.float32)
    init_ref[...] = jnp.dot(h, wr_ref[...],
                            preferred_element_type=jnp.float32) + b_ref[...]


def _mm2_kernel(i_ref, a_ref, w0_ref, w1_ref, w2_ref, wr_ref, b_ref,
                xr01_ref, xr2_ref, init_ref):
    h = jnp.maximum(i_ref[...] + a_ref[...], 0.0)
    xr01_ref[:, 0:HID] = jnp.dot(h, w0_ref[...],
                                 preferred_element_type=jnp.float32)
    xr01_ref[:, HID:2 * HID] = jnp.dot(h, w1_ref[...],
                                       preferred_element_type=jnp.float32)
    xr2_ref[...] = jnp.dot(h, w2_ref[...], preferred_element_type=jnp.float32)
    init_ref[...] = jnp.dot(h, wr_ref[...],
                            preferred_element_type=jnp.float32) + b_ref[...]


def _scatter_kernel(sr_ref, dst_ref, w_ref, xr01_ref, xr2p_ref, aggp_ref):
    aggp_ref[...] = jnp.zeros((N // 2, 2 * HID), jnp.float32)
    lane = jax.lax.broadcasted_iota(jnp.int32, (1, ECHUNK), 1)
    zero64 = jnp.zeros((1, HID), jnp.float32)

    def outer(i, _):
        srow = sr_ref[pl.ds(i, 1), :]
        drow = dst_ref[pl.ds(i, 1), :]
        wrow = w_ref[pl.ds(i, 1), :]

        def inner(j, _):
            m = lane == j
            sr = jnp.sum(jnp.where(m, srow, 0))
            d = jnp.sum(jnp.where(m, drow, 0))
            w = jnp.sum(jnp.where(m, wrow, 0.0))
            s = sr >> 2
            r = sr & 3
            m0 = xr01_ref[pl.ds(s, 1), 0:HID]
            m1 = xr01_ref[pl.ds(s, 1), HID:2 * HID]
            row2 = xr2p_ref[pl.ds(s >> 1, 1), :]
            m2 = jnp.where((s & 1) == 0, row2[:, 0:HID], row2[:, HID:2 * HID])
            msg = w * jnp.where(r == 0, m0, jnp.where(r == 1, m1, m2))
            upd = jnp.where((d & 1) == 0,
                            jnp.concatenate([msg, zero64], axis=1),
                            jnp.concatenate([zero64, msg], axis=1))
            d2 = d >> 1
            aggp_ref[pl.ds(d2, 1), :] = aggp_ref[pl.ds(d2, 1), :] + upd
            return 0

        return jax.lax.fori_loop(0, ECHUNK, inner, 0)

    jax.lax.fori_loop(0, EROWS, outer, 0)


def _pool_kernel(i_ref, a_ref, bat_ref, wl_ref, bl_ref,
                 acc_ref, cacc_ref, out_ref):
    pid = pl.program_id(0)

    @pl.when(pid == 0)
    def _():
        acc_ref[...] = jnp.zeros((NGRAPH, HID), jnp.float32)
        cacc_ref[...] = jnp.zeros((NGRAPH, 1), jnp.float32)

    h = jnp.maximum(i_ref[...] + a_ref[...], 0.0)
    lanes = jax.lax.broadcasted_iota(jnp.int32, (1, NGRAPH), 1)
    onehot = (bat_ref[...] == lanes).astype(jnp.float32)
    acc_ref[...] = acc_ref[...] + jnp.dot(onehot.T, h,
                                          preferred_element_type=jnp.float32)
    cacc_ref[...] = cacc_ref[...] + jnp.dot(
        onehot.T, jnp.ones((BLK, 1), jnp.float32),
        preferred_element_type=jnp.float32)

    @pl.when(pid == NBLK - 1)
    def _():
        pooled = acc_ref[...] / jnp.maximum(cacc_ref[...], 1.0)
        out_ref[...] = jnp.dot(pooled, wl_ref[...],
                               preferred_element_type=jnp.float32) + bl_ref[...]


def _conv_layer(first, h_or_init, agg_prev, W_rel, W_root, b,
                sr2, dst2, w2):
    din = W_root.shape[0]
    mmfn = _mm_kernel if first else _mm2_kernel
    ins = ([h_or_init] if first else [h_or_init, agg_prev])
    xr01, xr2, init = pl.pallas_call(
        mmfn,
        grid=(NBLK,),
        in_specs=[pl.BlockSpec((BLK, din), lambda i: (i, 0))] * len(ins)
        + [pl.BlockSpec((din, HID), lambda i: (0, 0))] * 4
        + [pl.BlockSpec((1, HID), lambda i: (0, 0))],
        out_specs=[
            pl.BlockSpec((BLK, 2 * HID), lambda i: (i, 0)),
            pl.BlockSpec((BLK, HID), lambda i: (i, 0)),
            pl.BlockSpec((BLK, HID), lambda i: (i, 0)),
        ],
        out_shape=[
            jax.ShapeDtypeStruct((N, 2 * HID), jnp.float32),
            jax.ShapeDtypeStruct((N, HID), jnp.float32),
            jax.ShapeDtypeStruct((N, HID), jnp.float32),
        ],
    )(*ins, W_rel[0], W_rel[1], W_rel[2], W_root, b.reshape(1, HID))

    xr2p = xr2.reshape(N // 2, 2 * HID)
    aggp = pl.pallas_call(
        _scatter_kernel,
        out_shape=jax.ShapeDtypeStruct((N // 2, 2 * HID), jnp.float32),
        compiler_params=pltpu.CompilerParams(
            vmem_limit_bytes=120 * 1024 * 1024),
    )(sr2, dst2, w2, xr01, xr2p)
    return init, aggp.reshape(N, HID)


def kernel(x, edge_index, edge_type, batch, shape_emb, color_emb, pos_emb,
           W1_rel, W1_root, b1, W2_rel, W2_root, b2, W_lin, b_lin):
    rel = edge_type.astype(jnp.int32)
    sr2 = (edge_index[0].astype(jnp.int32) * 4 + rel).reshape(EROWS, ECHUNK)
    dst2 = edge_index[1].reshape(EROWS, ECHUNK).astype(jnp.int32)
    rel2 = rel.reshape(EROWS, ECHUNK)
    bat2 = batch.reshape(N, 1).astype(jnp.int32)

    h = pl.pallas_call(
        _embed_kernel,
        grid=(NBLK,),
        in_specs=[
            pl.BlockSpec((BLK, 3), lambda i: (i, 0)),
            pl.BlockSpec((VOC, EMB), lambda i: (0, 0)),
            pl.BlockSpec((VOC, EMB), lambda i: (0, 0)),
            pl.BlockSpec((VOC, EMB), lambda i: (0, 0)),
        ],
        out_specs=pl.BlockSpec((BLK, 3 * EMB), lambda i: (i, 0)),
        out_shape=jax.ShapeDtypeStruct((N, 3 * EMB), jnp.float32),
    )(x.astype(jnp.int32), shape_emb, color_emb, pos_emb)

    w2 = pl.pallas_call(
        _wgt_kernel,
        out_shape=[
            jax.ShapeDtypeStruct((N, NREL), jnp.float32),
            jax.ShapeDtypeStruct((EROWS, ECHUNK), jnp.float32),
        ],
        compiler_params=pltpu.CompilerParams(
            vmem_limit_bytes=120 * 1024 * 1024),
    )(dst2, rel2)[1]

    init1, agg1 = _conv_layer(True, h, None, W1_rel, W1_root, b1,
                              sr2, dst2, w2)
    init2, agg2 = _conv_layer(False, init1, agg1, W2_rel, W2_root, b2,
                              sr2, dst2, w2)

    out = pl.pallas_call(
        _pool_kernel,
        grid=(NBLK,),
        in_specs=[
            pl.BlockSpec((BLK, HID), lambda i: (i, 0)),
            pl.BlockSpec((BLK, HID), lambda i: (i, 0)),
            pl.BlockSpec((BLK, 1), lambda i: (i, 0)),
            pl.BlockSpec((HID, NCLS), lambda i: (0, 0)),
            pl.BlockSpec((1, NCLS), lambda i: (0, 0)),
        ],
        out_specs=[
            pl.BlockSpec((NGRAPH, HID), lambda i: (0, 0)),
            pl.BlockSpec((NGRAPH, 1), lambda i: (0, 0)),
            pl.BlockSpec((NGRAPH, NCLS), lambda i: (0, 0)),
        ],
        out_shape=[
            jax.ShapeDtypeStruct((NGRAPH, HID), jnp.float32),
            jax.ShapeDtypeStruct((NGRAPH, 1), jnp.float32),
            jax.ShapeDtypeStruct((NGRAPH, NCLS), jnp.float32),
        ],
    )(init2, agg2, bat2, W_lin, b_lin.reshape(1, NCLS))[2]
    return out
